# Initial kernel scaffold; baseline (speedup 1.0000x reference)
import jax
import jax.numpy as jnp
from jax.experimental import pallas as pl


def kernel(x, Wg, bg, W1, b1, gamma, beta, W2, b2):
    raise NotImplementedError("write your pallas kernel here")


# speedup vs baseline: 2.5483x; 2.5483x over previous
import functools

import jax
import jax.numpy as jnp
from jax.experimental import pallas as pl
from jax.experimental.pallas import tpu as pltpu

D = 1024
E = 8
H = 512
C = 1000
CP = 1024
T = 4096
BT = 512


def _moe_block(x_ref, Wg_ref, bg_ref, W1_ref, b1_ref, g_ref, be_ref,
               W2_ref, b2_ref, logits_ref, out_ref):
    xb = x_ref[...]
    logits = xb @ Wg_ref[...] + bg_ref[...]
    logits_ref[...] = logits
    ei = jax.lax.broadcasted_iota(jnp.int32, logits.shape, 1)
    m0 = jnp.max(logits, axis=1, keepdims=True)
    i0 = jnp.min(jnp.where(logits == m0, ei, E), axis=1, keepdims=True)
    l2 = jnp.where(ei == i0, -jnp.inf, logits)
    m1 = jnp.max(l2, axis=1, keepdims=True)
    i1 = jnp.min(jnp.where(l2 == m1, ei, E), axis=1, keepdims=True)
    b = jnp.exp(m1 - m0)
    p0 = 1.0 / (1.0 + b)
    p1 = b / (1.0 + b)
    w = p0 * (ei == i0) + p1 * (ei == i1)

    acc = jnp.zeros((xb.shape[0], CP), jnp.float32)
    for e in range(E):
        h = xb @ W1_ref[e] + b1_ref[e]
        mu = jnp.mean(h, axis=-1, keepdims=True)
        var = jnp.mean((h - mu) ** 2, axis=-1, keepdims=True)
        hn = (h - mu) / jnp.sqrt(var + 1e-5) * g_ref[e] + be_ref[e]
        a = 0.5 * hn * (1.0 + jax.lax.erf(hn * 0.7071067811865476))
        acc = acc + w[:, e:e + 1] * (a @ W2_ref[e] + b2_ref[e])
    out_ref[...] = acc


@jax.jit
def kernel(x, Wg, bg, W1, b1, gamma, beta, W2, b2):
    W2p = jnp.pad(W2, ((0, 0), (0, 0), (0, CP - C)))
    b2p = jnp.pad(b2, ((0, 0), (0, CP - C)))
    grid = (T // BT,)
    logits, outp = pl.pallas_call(
        _moe_block,
        grid=grid,
        in_specs=[
            pl.BlockSpec((BT, D), lambda i: (i, 0)),
            pl.BlockSpec((D, E), lambda i: (0, 0)),
            pl.BlockSpec((1, E), lambda i: (0, 0)),
            pl.BlockSpec((E, D, H), lambda i: (0, 0, 0)),
            pl.BlockSpec((E, H), lambda i: (0, 0)),
            pl.BlockSpec((E, H), lambda i: (0, 0)),
            pl.BlockSpec((E, H), lambda i: (0, 0)),
            pl.BlockSpec((E, H, CP), lambda i: (0, 0, 0)),
            pl.BlockSpec((E, CP), lambda i: (0, 0)),
        ],
        out_specs=[
            pl.BlockSpec((BT, E), lambda i: (i, 0)),
            pl.BlockSpec((BT, CP), lambda i: (i, 0)),
        ],
        out_shape=[
            jax.ShapeDtypeStruct((T, E), jnp.float32),
            jax.ShapeDtypeStruct((T, CP), jnp.float32),
        ],
    )(x, Wg, bg.reshape(1, E), W1, b1, gamma, beta, W2p, b2p)
    return (outp[:, :C], logits)
